# baseline (device time: 150479 ns/iter reference)
import os

import jax
import jax.numpy as jnp
from jax import lax
from jax.experimental import pallas as pl
from jax.experimental.pallas import tpu as pltpu

_SKIP = set(os.environ.get("KERNEL_SKIP", "").split(","))

N_DEV = 16
N_SRC = N_DEV // 2


def kernel(x, Wq, K_ext, V_ext, Wo):
    B, Sq, D = x.shape
    _, Hq_loc_x_Dh = Wq.shape
    _, Skv_loc, H, Dh = K_ext.shape
    Hq_loc = Hq_loc_x_Dh // Dh
    R = B * Sq
    rows_per = R // N_DEV
    QB = Sq // 64
    Skv_sel = N_SRC * 64

    def body(x_ref, wq_ref, k_ref, v_ref, wo_ref, out_ref,
             stage, rbuf, pref, accbuf, redbuf,
             kv_recv, rs_recv, ag_recv, send_a, send_b):
        my = lax.axis_index("i")
        i_am_src = (my % 2) == 0
        my_slot = my // 2

        if "p1" not in _SKIP:
            @pl.when(i_am_src)
            def _():
                for d in range(N_DEV):
                    stage[d, 0] = k_ref[:, :, d * Hq_loc:(d + 1) * Hq_loc, :
                                        ].astype(jnp.bfloat16)
                    stage[d, 1] = v_ref[:, :, d * Hq_loc:(d + 1) * Hq_loc, :
                                        ].astype(jnp.bfloat16)

            kv_sends = []
            for o in range(1, N_DEV):
                d = (my + o) % N_DEV
                r = pltpu.make_async_remote_copy(
                    src_ref=stage.at[d],
                    dst_ref=rbuf.at[my_slot],
                    send_sem=send_a.at[d],
                    recv_sem=kv_recv.at[my],
                    device_id=(d,),
                    device_id_type=pl.DeviceIdType.MESH,
                )

                @pl.when(i_am_src)
                def _():
                    r.start()

                kv_sends.append(r)

            @pl.when(i_am_src)
            def _():
                rbuf[pl.ds(my_slot, 1)] = stage[pl.ds(my, 1)]

        wq = wq_ref[...]
        qs = []
        for b in range(B):
            qb_ = jnp.dot(x_ref[b], wq, preferred_element_type=jnp.float32)
            qs.append(qb_.reshape(Sq, Hq_loc, Dh))

        if "p1" not in _SKIP:
            for m in range(N_SRC):
                j = 2 * m
                r = pltpu.make_async_remote_copy(
                    src_ref=stage.at[0],
                    dst_ref=rbuf.at[m],
                    send_sem=send_a.at[j],
                    recv_sem=kv_recv.at[j],
                    device_id=(j,),
                    device_id_type=pl.DeviceIdType.MESH,
                )

                @pl.when(j != my)
                def _():
                    r.wait_recv()

            @pl.when(i_am_src)
            def _():
                for r in kv_sends:
                    r.wait_send()

        for b in range(B) if "p2" not in _SKIP else []:
            ctx_h = []
            for h in range(Hq_loc):
                ctx_q = []
                for qb in range(QB):
                    q = qs[b][qb * 64:(qb + 1) * 64, h, :]
                    kh = jnp.concatenate(
                        [rbuf[m, 0, b, qb * 64:(qb + 1) * 64, h, :]
                         for m in range(N_SRC)], axis=0,
                    ).astype(jnp.float32)
                    vh = jnp.concatenate(
                        [rbuf[m, 1, b, qb * 64:(qb + 1) * 64, h, :]
                         for m in range(N_SRC)], axis=0,
                    ).astype(jnp.float32)
                    s = jnp.dot(q, kh.T, preferred_element_type=jnp.float32)
                    s = s * 0.125
                    mx = jnp.max(s, axis=1, keepdims=True)
                    w = jnp.exp(s - mx)
                    w = w / jnp.sum(w, axis=1, keepdims=True)
                    ctx_q.append(
                        jnp.dot(w, vh, preferred_element_type=jnp.float32)
                    )
                ctx_h.append(jnp.concatenate(ctx_q, axis=0))
            ctx_b = jnp.concatenate(ctx_h, axis=1)
            pref[b * Sq:(b + 1) * Sq, :] = jnp.dot(
                ctx_b, wo_ref[...], preferred_element_type=jnp.float32
            )

        if "p2" in _SKIP:
            pref[...] = x_ref[...].reshape(R, D)

        if "p3" in _SKIP:
            out_ref[...] = pref[...].reshape(B, Sq, D)
            return

        rs_sends = []
        for o in range(1, N_DEV):
            d = (my + o) % N_DEV
            r = pltpu.make_async_remote_copy(
                src_ref=pref.at[pl.ds(d * rows_per, rows_per), :],
                dst_ref=accbuf.at[my],
                send_sem=send_a.at[d],
                recv_sem=rs_recv.at[my],
                device_id=(d,),
                device_id_type=pl.DeviceIdType.MESH,
            )
            r.start()
            rs_sends.append(r)
        accbuf[pl.ds(my, 1)] = pref[pl.ds(my * rows_per, rows_per), :][None]
        for o in range(1, N_DEV):
            j = (my + o) % N_DEV
            pltpu.make_async_remote_copy(
                src_ref=pref.at[pl.ds(0, rows_per), :],
                dst_ref=accbuf.at[j],
                send_sem=send_a.at[j],
                recv_sem=rs_recv.at[j],
                device_id=(j,),
                device_id_type=pl.DeviceIdType.MESH,
            ).wait_recv()
        for r in rs_sends:
            r.wait_send()

        reduced = jnp.sum(accbuf[...], axis=0)
        redbuf[...] = reduced

        my_b = my // (Sq // rows_per)
        my_row = (my % (Sq // rows_per)) * rows_per
        ag_sends = []
        for o in range(1, N_DEV):
            d = (my + o) % N_DEV
            r = pltpu.make_async_remote_copy(
                src_ref=redbuf,
                dst_ref=out_ref.at[my_b, pl.ds(my_row, rows_per), :],
                send_sem=send_b.at[d],
                recv_sem=ag_recv.at[my],
                device_id=(d,),
                device_id_type=pl.DeviceIdType.MESH,
            )
            r.start()
            ag_sends.append(r)
        out_ref[pl.ds(my_b, 1), pl.ds(my_row, rows_per), :] = reduced[None]
        for o in range(1, N_DEV):
            j = (my + o) % N_DEV
            jb = j // (Sq // rows_per)
            jrow = (j % (Sq // rows_per)) * rows_per
            pltpu.make_async_remote_copy(
                src_ref=redbuf,
                dst_ref=out_ref.at[jb, pl.ds(jrow, rows_per), :],
                send_sem=send_b.at[j],
                recv_sem=ag_recv.at[j],
                device_id=(j,),
                device_id_type=pl.DeviceIdType.MESH,
            ).wait_recv()
        for r in ag_sends:
            r.wait_send()

    return pl.pallas_call(
        body,
        out_shape=jax.ShapeDtypeStruct((B, Sq, D), jnp.float32),
        in_specs=[pl.BlockSpec(memory_space=pltpu.VMEM)] * 5,
        out_specs=pl.BlockSpec(memory_space=pltpu.VMEM),
        scratch_shapes=[
            pltpu.VMEM((N_DEV, 2, B, Skv_loc, Hq_loc, Dh), jnp.bfloat16),
            pltpu.VMEM((N_SRC, 2, B, Skv_loc, Hq_loc, Dh), jnp.bfloat16),
            pltpu.VMEM((R, D), jnp.float32),
            pltpu.VMEM((N_DEV, rows_per, D), jnp.float32),
            pltpu.VMEM((rows_per, D), jnp.float32),
            pltpu.SemaphoreType.DMA((N_DEV,)),
            pltpu.SemaphoreType.DMA((N_DEV,)),
            pltpu.SemaphoreType.DMA((N_DEV,)),
            pltpu.SemaphoreType.DMA((N_DEV,)),
            pltpu.SemaphoreType.DMA((N_DEV,)),
        ],
    )(x, Wq, K_ext, V_ext, Wo)
